# baseline (speedup 1.0000x reference)
import functools

import jax
import jax.numpy as jnp
from jax import lax
from jax.experimental import pallas as pl
from jax.experimental.pallas import tpu as pltpu
from jax.experimental.pallas import tpu_sc as plsc

SEQ_LEN = 8192
NUM_VALUES = 21
NUM_CORES = 1
NUM_SUBCORES = 16
LANES = 16
NUM_WORKERS = NUM_CORES * NUM_SUBCORES
ROWS_PER_TILE = SEQ_LEN // NUM_WORKERS
CHUNKS = ROWS_PER_TILE // LANES
QUARTERS = 2

_LN2 = 0.6931471805599453
_LN_COEFFS = (
    -0.017029610590466433, 0.1837008411638296, -0.8520795951885867,
    2.2269434608355745, -3.6471203953770273, 4.205234841506999,
    -2.0996478486876624,
)


def _vlog(x):
    bits = plsc.bitcast(x, jnp.int32)
    e = lax.shift_right_logical(bits, 23) - 127
    mbits = (bits & 0x007FFFFF) | 0x3F800000
    m = plsc.bitcast(mbits, jnp.float32)
    p = jnp.float32(_LN_COEFFS[0])
    for c in _LN_COEFFS[1:]:
        p = p * m + jnp.float32(c)
    return e.astype(jnp.float32) * _LN2 + p


_mesh = plsc.VectorSubcoreMesh(
    core_axis_name="c", subcore_axis_name="s",
    num_cores=NUM_CORES, num_subcores=NUM_SUBCORES,
)


@functools.partial(
    pl.kernel,
    out_type=jax.ShapeDtypeStruct((NUM_SUBCORES * LANES,), jnp.float32),
    mesh=_mesh,
    compiler_params=pltpu.CompilerParams(needs_layout_passes=False),
    scratch_types=[
        pltpu.VMEM((ROWS_PER_TILE,), jnp.int32),
        pltpu.VMEM((ROWS_PER_TILE * NUM_VALUES,), jnp.float32),
        pltpu.VMEM((LANES,), jnp.float32),
        pltpu.VMEM_SHARED((NUM_SUBCORES * LANES,), jnp.float32),
        pltpu.VMEM((NUM_SUBCORES * LANES,), jnp.float32),
        pltpu.VMEM((LANES,), jnp.float32),
        pltpu.SemaphoreType.DMA,
        pltpu.SemaphoreType.DMA,
        pltpu.SemaphoreType.DMA,
        pltpu.SemaphoreType.DMA,
        pltpu.SemaphoreType.DMA,
    ],
)
def _sc_logprob(seq_hbm, tab_hbm, out_hbm, seq_v, rows_v, acc_v,
                shared_sp, gath_v, res_v, sem_s, sem_q0, sem_q1, sem_q2,
                sem_q3):
    sid = lax.axis_index("s")
    base = sid * ROWS_PER_TILE
    qrows = ROWS_PER_TILE // QUARTERS
    qwords = qrows * NUM_VALUES
    cp_seq = pltpu.async_copy(
        seq_hbm.at[pl.ds(base, ROWS_PER_TILE)], seq_v, sem_s)
    cps = []
    for q, sem in enumerate((sem_q0, sem_q1, sem_q2, sem_q3)[:QUARTERS]):
        cps.append(pltpu.async_copy(
            tab_hbm.at[pl.ds(base * NUM_VALUES + q * qwords, qwords)],
            rows_v.at[pl.ds(q * qwords, qwords)], sem))
    cp_seq.wait()
    acc = jnp.zeros((LANES,), jnp.float32)
    for q in range(QUARTERS):
        cps[q].wait()
        for jq in range(CHUNKS // QUARTERS):
            j = q * (CHUNKS // QUARTERS) + jq
            cols = seq_v[pl.ds(j * LANES, LANES)]
            rows = lax.iota(jnp.int32, LANES) + (j * LANES)
            vals = plsc.load_gather(rows_v, [rows * NUM_VALUES + cols])
            acc = acc + _vlog(vals)
    acc_v[...] = acc
    pltpu.sync_copy(acc_v, out_hbm.at[pl.ds(sid * LANES, LANES)])


def kernel(sequence, site_probabilities):
    res = _sc_logprob(
        sequence.astype(jnp.int32), site_probabilities.reshape(-1))
    return -jnp.sum(res)


# speedup vs baseline: 1.1111x; 1.1111x over previous
import functools

import jax
import jax.numpy as jnp
from jax import lax
from jax.experimental import pallas as pl
from jax.experimental.pallas import tpu as pltpu
from jax.experimental.pallas import tpu_sc as plsc

SEQ_LEN = 8192
NUM_VALUES = 21
NUM_CORES = 1
NUM_SUBCORES = 16
LANES = 16
NUM_WORKERS = NUM_CORES * NUM_SUBCORES
ROWS_PER_TILE = SEQ_LEN // NUM_WORKERS
CHUNKS = ROWS_PER_TILE // LANES
QUARTERS = 2

_LN2 = 0.6931471805599453
_LN_COEFFS = (
    -0.017029610590466433, 0.1837008411638296, -0.8520795951885867,
    2.2269434608355745, -3.6471203953770273, 4.205234841506999,
    -2.0996478486876624,
)


def _vlog(x):
    bits = plsc.bitcast(x, jnp.int32)
    e = lax.shift_right_logical(bits, 23) - 127
    mbits = (bits & 0x007FFFFF) | 0x3F800000
    m = plsc.bitcast(mbits, jnp.float32)
    p = jnp.float32(_LN_COEFFS[0])
    for c in _LN_COEFFS[1:]:
        p = p * m + jnp.float32(c)
    return e.astype(jnp.float32) * _LN2 + p


_mesh = plsc.VectorSubcoreMesh(
    core_axis_name="c", subcore_axis_name="s",
    num_cores=NUM_CORES, num_subcores=NUM_SUBCORES,
)


@functools.partial(
    pl.kernel,
    out_type=jax.ShapeDtypeStruct((1,), jnp.float32),
    mesh=_mesh,
    compiler_params=pltpu.CompilerParams(needs_layout_passes=False),
    scratch_types=[
        pltpu.VMEM((ROWS_PER_TILE,), jnp.int32),
        pltpu.VMEM((ROWS_PER_TILE, NUM_VALUES), jnp.float32),
        pltpu.VMEM((LANES,), jnp.float32),
        pltpu.VMEM_SHARED((NUM_SUBCORES * LANES,), jnp.float32),
        pltpu.VMEM((NUM_SUBCORES * LANES,), jnp.float32),
        pltpu.VMEM((LANES,), jnp.float32),
        pltpu.SemaphoreType.DMA,
        pltpu.SemaphoreType.DMA,
        pltpu.SemaphoreType.DMA,
        pltpu.SemaphoreType.DMA,
        pltpu.SemaphoreType.DMA,
    ],
)
def _sc_logprob(seq_hbm, tab_hbm, out_hbm, seq_v, rows_v, acc_v,
                shared_sp, gath_v, res_v, sem_s, sem_q0, sem_q1, sem_q2,
                sem_q3):
    sid = lax.axis_index("s")
    base = sid * ROWS_PER_TILE
    qrows = ROWS_PER_TILE // QUARTERS
    qwords = qrows * NUM_VALUES
    cp_seq = pltpu.async_copy(
        seq_hbm.at[pl.ds(base, ROWS_PER_TILE)], seq_v, sem_s)
    cps = []
    for q, sem in enumerate((sem_q0, sem_q1, sem_q2, sem_q3)[:QUARTERS]):
        cps.append(pltpu.async_copy(
            tab_hbm.at[pl.ds(base + q * qrows, qrows)],
            rows_v.at[pl.ds(q * qrows, qrows)], sem))
    cp_seq.wait()
    acc = jnp.zeros((LANES,), jnp.float32)
    for q in range(QUARTERS):
        cps[q].wait()
        for jq in range(CHUNKS // QUARTERS):
            j = q * (CHUNKS // QUARTERS) + jq
            cols = seq_v[pl.ds(j * LANES, LANES)]
            rows = lax.iota(jnp.int32, LANES) + (j * LANES)
            vals = plsc.load_gather(rows_v, [rows, cols])
            acc = acc + _vlog(vals)
    acc_v[...] = acc
    pltpu.sync_copy(acc_v, shared_sp.at[pl.ds(sid * LANES, LANES)])
    plsc.subcore_barrier()

    @pl.when(sid == 0)
    def _():
        pltpu.sync_copy(shared_sp, gath_v)
        total = jnp.zeros((LANES,), jnp.float32)
        for k in range(NUM_SUBCORES):
            total = total + gath_v[pl.ds(k * LANES, LANES)]
        res_v[...] = jnp.broadcast_to(-jnp.sum(total), (LANES,))
        pltpu.sync_copy(res_v.at[pl.ds(0, 1)], out_hbm)


def kernel(sequence, site_probabilities):
    res = _sc_logprob(sequence.astype(jnp.int32), site_probabilities)
    return res.reshape(())


# speedup vs baseline: 1.1162x; 1.0046x over previous
import functools

import jax
import jax.numpy as jnp
from jax import lax
from jax.experimental import pallas as pl
from jax.experimental.pallas import tpu as pltpu
from jax.experimental.pallas import tpu_sc as plsc

SEQ_LEN = 8192
NUM_VALUES = 21
NUM_CORES = 1
NUM_SUBCORES = 16
LANES = 16
NUM_WORKERS = NUM_CORES * NUM_SUBCORES
ROWS_PER_TILE = SEQ_LEN // NUM_WORKERS
CHUNKS = ROWS_PER_TILE // LANES
QUARTERS = 2

_LN2 = 0.6931471805599453
_LN_COEFFS = (
    -0.017029610590466433, 0.1837008411638296, -0.8520795951885867,
    2.2269434608355745, -3.6471203953770273, 4.205234841506999,
    -2.0996478486876624,
)


def _vlog_parts(x):
    bits = plsc.bitcast(x, jnp.int32)
    e_raw = lax.shift_right_logical(bits, 23)
    mbits = (bits & 0x007FFFFF) | 0x3F800000
    m = plsc.bitcast(mbits, jnp.float32)
    p = jnp.float32(_LN_COEFFS[0])
    for c in _LN_COEFFS[1:]:
        p = p * m + jnp.float32(c)
    return p, e_raw


_mesh = plsc.VectorSubcoreMesh(
    core_axis_name="c", subcore_axis_name="s",
    num_cores=NUM_CORES, num_subcores=NUM_SUBCORES,
)


@functools.partial(
    pl.kernel,
    out_type=jax.ShapeDtypeStruct((1,), jnp.float32),
    mesh=_mesh,
    compiler_params=pltpu.CompilerParams(needs_layout_passes=False),
    scratch_types=[
        pltpu.VMEM((ROWS_PER_TILE,), jnp.int32),
        pltpu.VMEM((ROWS_PER_TILE, NUM_VALUES), jnp.float32),
        pltpu.VMEM((LANES,), jnp.float32),
        pltpu.VMEM_SHARED((NUM_SUBCORES * LANES,), jnp.float32),
        pltpu.VMEM((NUM_SUBCORES * LANES,), jnp.float32),
        pltpu.VMEM((LANES,), jnp.float32),
        pltpu.SemaphoreType.DMA,
        pltpu.SemaphoreType.DMA,
        pltpu.SemaphoreType.DMA,
        pltpu.SemaphoreType.DMA,
        pltpu.SemaphoreType.DMA,
    ],
)
def _sc_logprob(seq_hbm, tab_hbm, out_hbm, seq_v, rows_v, acc_v,
                shared_sp, gath_v, res_v, sem_s, sem_q0, sem_q1, sem_q2,
                sem_q3):
    sid = lax.axis_index("s")
    base = sid * ROWS_PER_TILE
    qrows = ROWS_PER_TILE // QUARTERS
    qwords = qrows * NUM_VALUES
    cp_seq = pltpu.async_copy(
        seq_hbm.at[pl.ds(base, ROWS_PER_TILE)], seq_v, sem_s)
    cps = []
    for q, sem in enumerate((sem_q0, sem_q1, sem_q2, sem_q3)[:QUARTERS]):
        cps.append(pltpu.async_copy(
            tab_hbm.at[pl.ds(base + q * qrows, qrows)],
            rows_v.at[pl.ds(q * qrows, qrows)], sem))
    cp_seq.wait()
    acc_f = jnp.zeros((LANES,), jnp.float32)
    acc_e = jnp.zeros((LANES,), jnp.int32)
    for q in range(QUARTERS):
        cps[q].wait()
        for jq in range(CHUNKS // QUARTERS):
            j = q * (CHUNKS // QUARTERS) + jq
            cols = seq_v[pl.ds(j * LANES, LANES)]
            rows = lax.iota(jnp.int32, LANES) + (j * LANES)
            vals = plsc.load_gather(rows_v, [rows, cols])
            p, e_raw = _vlog_parts(vals)
            acc_f = acc_f + p
            acc_e = acc_e + e_raw
    acc_v[...] = acc_f + (acc_e.astype(jnp.float32) - (127.0 * CHUNKS)) * _LN2
    pltpu.sync_copy(acc_v, shared_sp.at[pl.ds(sid * LANES, LANES)])
    plsc.subcore_barrier()

    @pl.when(sid == 0)
    def _():
        pltpu.sync_copy(shared_sp, gath_v)
        total = jnp.zeros((LANES,), jnp.float32)
        for k in range(NUM_SUBCORES):
            total = total + gath_v[pl.ds(k * LANES, LANES)]
        res_v[...] = jnp.broadcast_to(-jnp.sum(total), (LANES,))
        pltpu.sync_copy(res_v.at[pl.ds(0, 1)], out_hbm)


def kernel(sequence, site_probabilities):
    res = _sc_logprob(sequence.astype(jnp.int32), site_probabilities)
    return res.reshape(())


# speedup vs baseline: 1.1189x; 1.0024x over previous
import functools

import jax
import jax.numpy as jnp
from jax import lax
from jax.experimental import pallas as pl
from jax.experimental.pallas import tpu as pltpu
from jax.experimental.pallas import tpu_sc as plsc

SEQ_LEN = 8192
NUM_VALUES = 21
NUM_CORES = 1
NUM_SUBCORES = 16
LANES = 16
NUM_WORKERS = NUM_CORES * NUM_SUBCORES
ROWS_PER_TILE = SEQ_LEN // NUM_WORKERS
CHUNKS = ROWS_PER_TILE // LANES
QUARTERS = 2

_LN2 = 0.6931471805599453
_LN_COEFFS = (
    -0.017029610590466433, 0.1837008411638296, -0.8520795951885867,
    2.2269434608355745, -3.6471203953770273, 4.205234841506999,
    -2.0996478486876624,
)


def _vlog_parts(x):
    bits = plsc.bitcast(x, jnp.int32)
    e_raw = lax.shift_right_logical(bits, 23)
    mbits = (bits & 0x007FFFFF) | 0x3F800000
    m = plsc.bitcast(mbits, jnp.float32)
    p = jnp.float32(_LN_COEFFS[0])
    for c in _LN_COEFFS[1:]:
        p = p * m + jnp.float32(c)
    return p, e_raw


_mesh = plsc.VectorSubcoreMesh(
    core_axis_name="c", subcore_axis_name="s",
    num_cores=NUM_CORES, num_subcores=NUM_SUBCORES,
)


@functools.partial(
    pl.kernel,
    out_type=jax.ShapeDtypeStruct((1,), jnp.float32),
    mesh=_mesh,
    compiler_params=pltpu.CompilerParams(needs_layout_passes=False),
    scratch_types=[
        pltpu.VMEM((ROWS_PER_TILE,), jnp.int32),
        pltpu.VMEM((ROWS_PER_TILE, NUM_VALUES), jnp.float32),
        pltpu.VMEM((LANES,), jnp.float32),
        pltpu.VMEM_SHARED((NUM_SUBCORES * 8,), jnp.float32),
        pltpu.VMEM((NUM_SUBCORES * 8,), jnp.float32),
        pltpu.VMEM((LANES,), jnp.float32),
        pltpu.SemaphoreType.DMA,
        pltpu.SemaphoreType.DMA,
        pltpu.SemaphoreType.DMA,
        pltpu.SemaphoreType.DMA,
        pltpu.SemaphoreType.DMA,
    ],
)
def _sc_logprob(seq_hbm, tab_hbm, out_hbm, seq_v, rows_v, acc_v,
                shared_sp, gath_v, res_v, sem_s, sem_q0, sem_q1, sem_q2,
                sem_q3):
    sid = lax.axis_index("s")
    base = sid * ROWS_PER_TILE
    qrows = ROWS_PER_TILE // QUARTERS
    qwords = qrows * NUM_VALUES
    cp_seq = pltpu.async_copy(
        seq_hbm.at[pl.ds(base, ROWS_PER_TILE)], seq_v, sem_s)
    cps = []
    for q, sem in enumerate((sem_q0, sem_q1, sem_q2, sem_q3)[:QUARTERS]):
        cps.append(pltpu.async_copy(
            tab_hbm.at[pl.ds(base + q * qrows, qrows)],
            rows_v.at[pl.ds(q * qrows, qrows)], sem))
    cp_seq.wait()
    acc_f = jnp.zeros((LANES,), jnp.float32)
    acc_e = jnp.zeros((LANES,), jnp.int32)
    for q in range(QUARTERS):
        cps[q].wait()
        for jq in range(CHUNKS // QUARTERS):
            j = q * (CHUNKS // QUARTERS) + jq
            cols = seq_v[pl.ds(j * LANES, LANES)]
            rows = lax.iota(jnp.int32, LANES) + (j * LANES)
            vals = plsc.load_gather(rows_v, [rows, cols])
            p, e_raw = _vlog_parts(vals)
            acc_f = acc_f + p
            acc_e = acc_e + e_raw
    tile_total = jnp.sum(
        acc_f + (acc_e.astype(jnp.float32) - (127.0 * CHUNKS)) * _LN2)
    acc_v[...] = jnp.broadcast_to(tile_total, (LANES,))
    pltpu.sync_copy(acc_v.at[pl.ds(0, 8)], shared_sp.at[pl.ds(sid * 8, 8)])
    plsc.subcore_barrier()

    @pl.when(sid == 0)
    def _():
        pltpu.sync_copy(shared_sp, gath_v)
        total = jnp.zeros((LANES,), jnp.float32)
        for k in range(NUM_SUBCORES // 2):
            total = total + gath_v[pl.ds(k * LANES, LANES)]
        res_v[...] = jnp.broadcast_to(jnp.float32(-0.125) * jnp.sum(total),
                                      (LANES,))
        pltpu.sync_copy(res_v.at[pl.ds(0, 1)], out_hbm)


def kernel(sequence, site_probabilities):
    res = _sc_logprob(sequence.astype(jnp.int32), site_probabilities)
    return res.reshape(())


# speedup vs baseline: 1.1202x; 1.0012x over previous
import functools

import jax
import jax.numpy as jnp
from jax import lax
from jax.experimental import pallas as pl
from jax.experimental.pallas import tpu as pltpu
from jax.experimental.pallas import tpu_sc as plsc

SEQ_LEN = 8192
NUM_VALUES = 21
NUM_CORES = 1
NUM_SUBCORES = 16
LANES = 16
NUM_WORKERS = NUM_CORES * NUM_SUBCORES
ROWS_PER_TILE = SEQ_LEN // NUM_WORKERS
CHUNKS = ROWS_PER_TILE // LANES
QUARTERS = 2

_LN2 = 0.6931471805599453
_LN_COEFFS = (
    -0.017029610590466433, 0.1837008411638296, -0.8520795951885867,
    2.2269434608355745, -3.6471203953770273, 4.205234841506999,
    -2.0996478486876624,
)


def _split(x):
    bits = plsc.bitcast(x, jnp.int32)
    e_raw = lax.shift_right_logical(bits, 23)
    mbits = (bits & 0x007FFFFF) | 0x3F800000
    return plsc.bitcast(mbits, jnp.float32), e_raw


def _vlog_parts(x):
    m, e_raw = _split(x)
    p = jnp.float32(_LN_COEFFS[0])
    for c in _LN_COEFFS[1:]:
        p = p * m + jnp.float32(c)
    return p, e_raw


_mesh = plsc.VectorSubcoreMesh(
    core_axis_name="c", subcore_axis_name="s",
    num_cores=NUM_CORES, num_subcores=NUM_SUBCORES,
)


@functools.partial(
    pl.kernel,
    out_type=jax.ShapeDtypeStruct((1,), jnp.float32),
    mesh=_mesh,
    compiler_params=pltpu.CompilerParams(needs_layout_passes=False),
    scratch_types=[
        pltpu.VMEM((ROWS_PER_TILE,), jnp.int32),
        pltpu.VMEM((ROWS_PER_TILE, NUM_VALUES), jnp.float32),
        pltpu.VMEM((LANES,), jnp.float32),
        pltpu.VMEM_SHARED((NUM_SUBCORES * 8,), jnp.float32),
        pltpu.VMEM((NUM_SUBCORES * 8,), jnp.float32),
        pltpu.VMEM((LANES,), jnp.float32),
        pltpu.SemaphoreType.DMA,
        pltpu.SemaphoreType.DMA,
        pltpu.SemaphoreType.DMA,
        pltpu.SemaphoreType.DMA,
        pltpu.SemaphoreType.DMA,
    ],
)
def _sc_logprob(seq_hbm, tab_hbm, out_hbm, seq_v, rows_v, acc_v,
                shared_sp, gath_v, res_v, sem_s, sem_q0, sem_q1, sem_q2,
                sem_q3):
    sid = lax.axis_index("s")
    base = sid * ROWS_PER_TILE
    qrows = ROWS_PER_TILE // QUARTERS
    qwords = qrows * NUM_VALUES
    cp_seq = pltpu.async_copy(
        seq_hbm.at[pl.ds(base, ROWS_PER_TILE)], seq_v, sem_s)
    cps = []
    for q, sem in enumerate((sem_q0, sem_q1, sem_q2, sem_q3)[:QUARTERS]):
        cps.append(pltpu.async_copy(
            tab_hbm.at[pl.ds(base + q * qrows, qrows)],
            rows_v.at[pl.ds(q * qrows, qrows)], sem))
    cp_seq.wait()
    acc_f = jnp.zeros((LANES,), jnp.float32)
    acc_e = jnp.zeros((LANES,), jnp.int32)
    lane_iota = lax.iota(jnp.int32, LANES)
    for q in range(QUARTERS):
        cps[q].wait()
        for jp in range(CHUNKS // QUARTERS // 2):
            j = q * (CHUNKS // QUARTERS) + 2 * jp
            cols0 = seq_v[pl.ds(j * LANES, LANES)]
            rows0 = lane_iota + (j * LANES)
            vals0 = plsc.load_gather(rows_v, [rows0, cols0])
            cols1 = seq_v[pl.ds((j + 1) * LANES, LANES)]
            rows1 = lane_iota + ((j + 1) * LANES)
            vals1 = plsc.load_gather(rows_v, [rows1, cols1])
            m0, e0 = _split(vals0)
            m1, e1 = _split(vals1)
            p, eq = _vlog_parts(m0 * m1)
            acc_f = acc_f + p
            acc_e = acc_e + (e0 + e1 + eq)
    tile_total = jnp.sum(
        acc_f
        + (acc_e.astype(jnp.float32) - (127.0 * 3 * (CHUNKS // 2))) * _LN2)
    acc_v[...] = jnp.broadcast_to(tile_total, (LANES,))
    pltpu.sync_copy(acc_v.at[pl.ds(0, 8)], shared_sp.at[pl.ds(sid * 8, 8)])
    plsc.subcore_barrier()

    @pl.when(sid == 0)
    def _():
        pltpu.sync_copy(shared_sp, gath_v)
        total = jnp.zeros((LANES,), jnp.float32)
        for k in range(NUM_SUBCORES // 2):
            total = total + gath_v[pl.ds(k * LANES, LANES)]
        res_v[...] = jnp.broadcast_to(jnp.float32(-0.125) * jnp.sum(total),
                                      (LANES,))
        pltpu.sync_copy(res_v.at[pl.ds(0, 1)], out_hbm)


def kernel(sequence, site_probabilities):
    res = _sc_logprob(sequence.astype(jnp.int32), site_probabilities)
    return res.reshape(())
